# Initial kernel scaffold; baseline (speedup 1.0000x reference)
import jax
import jax.numpy as jnp
from jax.experimental import pallas as pl


def kernel(img, enc_w1, enc_b1, enc_w2, enc_b2, codebook, dec_w1, dec_b1, dec_w2, dec_b2):
    raise NotImplementedError("write your pallas kernel here")


# speedup vs baseline: 1.1823x; 1.1823x over previous
import functools

import jax
import jax.numpy as jnp
from jax.experimental import pallas as pl
from jax.experimental.pallas import tpu as pltpu

LATENT_DIM = 64
HIDDEN = 128
NUM_EMB = 1024
BETA = 0.25

ROW_BLK = 512


def _vq_loss_kernel(z_ref, cb_ref, cb2_ref, out_ref, acc_ref):
    i = pl.program_id(0)
    z = z_ref[...]
    cb = cb_ref[...]
    xc = jax.lax.dot_general(z, cb, (((1,), (1,)), ((), ())),
                             preferred_element_type=jnp.float32)
    d = cb2_ref[...] - 2.0 * xc
    m = jnp.min(d, axis=1)
    z2 = jnp.sum(z * z, axis=1)
    partial = jnp.sum(m + z2)

    @pl.when(i == 0)
    def _init():
        acc_ref[0, 0] = 0.0

    acc_ref[0, 0] += partial

    @pl.when(i == pl.num_programs(0) - 1)
    def _fin():
        out_ref[0, 0] = acc_ref[0, 0]


def _vq_loss(z_flat, codebook):
    n_rows = z_flat.shape[0]
    grid = n_rows // ROW_BLK
    cb2 = jnp.sum(codebook * codebook, axis=1)[None, :]
    total = pl.pallas_call(
        _vq_loss_kernel,
        grid=(grid,),
        in_specs=[
            pl.BlockSpec((ROW_BLK, LATENT_DIM), lambda i: (i, 0)),
            pl.BlockSpec((NUM_EMB, LATENT_DIM), lambda i: (0, 0)),
            pl.BlockSpec((1, NUM_EMB), lambda i: (0, 0)),
        ],
        out_specs=pl.BlockSpec((1, 1), lambda i: (0, 0), memory_space=pltpu.SMEM),
        out_shape=jax.ShapeDtypeStruct((1, 1), jnp.float32),
        scratch_shapes=[pltpu.SMEM((1, 1), jnp.float32)],
    )(z_flat, codebook, cb2)
    return total[0, 0]


def _conv(x, w, b, stride):
    y = jax.lax.conv_general_dilated(
        x, w, window_strides=(stride, stride), padding=((1, 1), (1, 1)),
        dimension_numbers=('NCHW', 'OIHW', 'NCHW'))
    return y + b[None, :, None, None]


def _conv_transpose(x, w, b):
    w_flip = w[:, :, ::-1, ::-1]
    y = jax.lax.conv_general_dilated(
        x, w_flip, window_strides=(1, 1), padding=((2, 2), (2, 2)),
        lhs_dilation=(2, 2), dimension_numbers=('NCHW', 'OIHW', 'NCHW'))
    return y + b[None, :, None, None]


def kernel(img, enc_w1, enc_b1, enc_w2, enc_b2, codebook, dec_w1, dec_b1, dec_w2, dec_b2):
    h = jax.nn.relu(_conv(img, enc_w1, enc_b1, 2))
    z = _conv(h, enc_w2, enc_b2, 2)
    B, C, H, W = z.shape
    z_flat = z.reshape(B, C, H * W).transpose(0, 2, 1).reshape(B * H * W, C)
    total = _vq_loss(z_flat, codebook)
    loss = (1.0 + BETA) * total / (B * C * H * W)
    hd = jax.nn.relu(_conv_transpose(z, dec_w1, dec_b1))
    out = _conv_transpose(hd, dec_w2, dec_b2)
    return (out, loss)


# speedup vs baseline: 1.2416x; 1.0501x over previous
import jax
import jax.numpy as jnp
from jax.experimental import pallas as pl
from jax.experimental.pallas import tpu as pltpu

LATENT_DIM = 64
HIDDEN = 128
NUM_EMB = 1024
BETA = 0.25


def _vq_loss_kernel(z_ref, cb_ref, cb2_ref, out_ref, acc_ref):
    b = pl.program_id(0)
    z = z_ref[0]
    cb = cb_ref[...]
    xc = jax.lax.dot_general(cb, z, (((1,), (0,)), ((), ())),
                             preferred_element_type=jnp.float32)
    d = cb2_ref[...] - 2.0 * xc
    m = jnp.min(d, axis=0)
    z2 = jnp.sum(z * z, axis=0)
    partial = jnp.sum(m + z2)

    @pl.when(b == 0)
    def _init():
        acc_ref[0, 0] = 0.0

    acc_ref[0, 0] += partial

    @pl.when(b == pl.num_programs(0) - 1)
    def _fin():
        out_ref[0, 0] = acc_ref[0, 0]


def _vq_loss(z, codebook):
    B, C, HW = z.shape
    cb2 = jnp.sum(codebook * codebook, axis=1)[:, None]
    total = pl.pallas_call(
        _vq_loss_kernel,
        grid=(B,),
        in_specs=[
            pl.BlockSpec((1, C, HW), lambda b: (b, 0, 0)),
            pl.BlockSpec((NUM_EMB, C), lambda b: (0, 0)),
            pl.BlockSpec((NUM_EMB, 1), lambda b: (0, 0)),
        ],
        out_specs=pl.BlockSpec((1, 1), lambda b: (0, 0), memory_space=pltpu.SMEM),
        out_shape=jax.ShapeDtypeStruct((1, 1), jnp.float32),
        scratch_shapes=[pltpu.SMEM((1, 1), jnp.float32)],
    )(z, codebook, cb2)
    return total[0, 0]


def _conv(x, w, b, stride):
    y = jax.lax.conv_general_dilated(
        x, w, window_strides=(stride, stride), padding=((1, 1), (1, 1)),
        dimension_numbers=('NCHW', 'OIHW', 'NCHW'))
    return y + b[None, :, None, None]


def _conv_transpose(x, w, b):
    w_flip = w[:, :, ::-1, ::-1]
    y = jax.lax.conv_general_dilated(
        x, w_flip, window_strides=(1, 1), padding=((2, 2), (2, 2)),
        lhs_dilation=(2, 2), dimension_numbers=('NCHW', 'OIHW', 'NCHW'))
    return y + b[None, :, None, None]


def kernel(img, enc_w1, enc_b1, enc_w2, enc_b2, codebook, dec_w1, dec_b1, dec_w2, dec_b2):
    h = jax.nn.relu(_conv(img, enc_w1, enc_b1, 2))
    z = _conv(h, enc_w2, enc_b2, 2)
    B, C, H, W = z.shape
    total = _vq_loss(z.reshape(B, C, H * W), codebook)
    loss = (1.0 + BETA) * total / (B * C * H * W)
    hd = jax.nn.relu(_conv_transpose(z, dec_w1, dec_b1))
    out = _conv_transpose(hd, dec_w2, dec_b2)
    return (out, loss)
